# Initial kernel scaffold; baseline (speedup 1.0000x reference)
import jax
import jax.numpy as jnp
from jax.experimental import pallas as pl


def kernel(log_snr, input_ids):
    raise NotImplementedError("write your pallas kernel here")


# speedup vs baseline: 7.0848x; 7.0848x over previous
import jax
import jax.numpy as jnp
from jax.experimental import pallas as pl
from jax.experimental.pallas import tpu as pltpu

VOCAB = 32768
MASK = 32767
BV = 4096


def _fill_block(ls_ref, ids_ref, out_ref):
    i = pl.program_id(0)
    j = pl.program_id(1)
    ls = ls_ref[i, :]
    ids = ids_ref[i, :]
    L = ls.shape[0]
    alpha = jax.nn.sigmoid(jnp.clip(ls, -10.0, 10.0))
    col = jax.lax.broadcasted_iota(jnp.int32, (L, BV), 1)
    ids_adj = ids - j * BV
    out_ref[...] = jnp.where(col == ids_adj[:, None], alpha[:, None], 0.0)

    @pl.when(j == pl.num_programs(1) - 1)
    def _():
        mask_val = (1.0 - alpha) + jnp.where(ids == MASK, alpha, 0.0)
        out_ref[:, BV - 1:BV] = mask_val[:, None]


def kernel(log_snr, input_ids):
    B, L = log_snr.shape
    out = pl.pallas_call(
        _fill_block,
        grid=(B, VOCAB // BV),
        in_specs=[
            pl.BlockSpec((B, L), lambda i, j: (0, 0)),
            pl.BlockSpec((B, L), lambda i, j: (0, 0)),
        ],
        out_specs=pl.BlockSpec((L, BV), lambda i, j: (i, j)),
        out_shape=jax.ShapeDtypeStruct((B * L, VOCAB), jnp.float32),
        compiler_params=pltpu.CompilerParams(
            dimension_semantics=("parallel", "arbitrary"),
        ),
    )(log_snr, input_ids.astype(jnp.int32))
    return out.reshape(B, L, VOCAB)
